# baseline (speedup 1.0000x reference)
import functools

import jax
import jax.numpy as jnp
from jax import lax
from jax.experimental import pallas as pl
from jax.experimental.pallas import tpu as pltpu
from jax.experimental.pallas import tpu_sc as plsc

BETA = 5.0
XY_WEIGHT = 0.5
B, M, DSLOT, C, K, D = 64, 256, 126, 100, 32, 128
CK = C * K
LOG2E = 1.4426950408889634
NB = 4
NG = 4
NJ = 8
MJ = M // NJ


def _make_sc_body(G):
    TPG = 32 // G
    MJt = M // TPG

    def _topp_sc_body(p_hbm, m_hbm, out_hbm, pv, mv, csv, cnv):
        tile = lax.axis_index("c") * 16 + lax.axis_index("s")
        grp = tile // TPG
        j0 = (tile % TPG) * MJt
        pltpu.sync_copy(p_hbm.at[grp], pv)
        pltpu.sync_copy(m_hbm.at[grp], mv)
        zeros = jnp.zeros((16,), jnp.float32)

        def _smul(j, carry):
            pv[j] = pv[j] * mv[j]
            return carry
        lax.fori_loop(0, M, _smul, zeros)

        JB = 4

        def _jbody(jb, carry):
            j = j0 + jb * JB
            sjs = [pv[j + q] for q in range(JB)]

            def _lbody(l, acc):
                sl = pv[l]
                new = []
                for q in range(JB):
                    cs, cn = acc[2 * q], acc[2 * q + 1]
                    lef = jnp.where(l <= j + q, 1.0, 0.0)
                    m = jnp.where(sl > sjs[q], 1.0,
                                  jnp.where(sl == sjs[q], lef, 0.0))
                    new += [cs + m * sl, cn + m]
                return tuple(new)

            acc = lax.fori_loop(0, M, _lbody, (zeros,) * (2 * JB), unroll=4)
            for q in range(JB):
                csv[jb * JB + q] = acc[2 * q]
                cnv[jb * JB + q] = acc[2 * q + 1]
            return carry

        lax.fori_loop(0, MJt // JB, _jbody, zeros)
        pltpu.sync_copy(csv, out_hbm.at[grp, pl.ds(j0, MJt)])
        pltpu.sync_copy(cnv, out_hbm.at[grp, pl.ds(M + j0, MJt)])

    return _topp_sc_body


def _topp_sc(pg, mg):
    f32 = jnp.float32
    G = pg.shape[0]
    MJt = M // (32 // G)
    fn = functools.partial(
        pl.kernel,
        mesh=plsc.VectorSubcoreMesh(core_axis_name="c", subcore_axis_name="s"),
        out_type=jax.ShapeDtypeStruct((G, 2 * M, 16), f32),
        scratch_types=[
            pltpu.VMEM((M, 16), f32),
            pltpu.VMEM((M, 16), f32),
            pltpu.VMEM((MJt, 16), f32),
            pltpu.VMEM((MJt, 16), f32),
        ],
    )(_make_sc_body(G))
    return fn(pg, mg)


def _prep_body(cf_ref, psif_ref, cft_out, cb_out, sel_out):
    cf = cf_ref[...]
    cft_out[...] = cf.T
    cft = cft_out[...]
    c2 = jnp.dot(jnp.ones((1, D), jnp.float32), cft * cft,
                 preferred_element_type=jnp.float32)
    seg = jax.lax.broadcasted_iota(jnp.int32, (CK, D), 0) // K
    cidx = jax.lax.broadcasted_iota(jnp.int32, (CK, D), 1)
    sel = (seg == cidx).astype(jnp.float32)
    sel_out[...] = sel
    psif = psif_ref[...]
    mg = jnp.max(psif)
    e = jnp.exp(psif - mg)
    seg_sum = jnp.dot(e, sel, preferred_element_type=jnp.float32)
    sums = jax.lax.dot_general(seg_sum, sel, (((1,), (1,)), ((), ())),
                               preferred_element_type=jnp.float32)
    lnw = psif - mg - jnp.log(sums)
    cb_out[...] = LOG2E * (lnw - BETA * c2)


def _mixer_kernel(tp_ref, ap_ref, s_ref, xy_ref, cr_ref, base_ref, cf_ref,
                  psif_ref, out_ref, cft_ref, cb_ref, sel_ref):
    @pl.when(pl.program_id(0) == 0)
    def _prep():
        _prep_body(cf_ref, psif_ref, cft_ref, cb_ref, sel_ref)

    MM = NB * M
    s_in = s_ref[...].reshape(MM, DSLOT)
    xy_in = xy_ref[...].reshape(MM, 2)
    sxy = jnp.concatenate([s_in, xy_in], axis=-1)
    xsq = sxy * sxy
    lane = jax.lax.broadcasted_iota(jnp.int32, (MM, D), 1)
    is_s = lane < DSLOT
    n1 = jnp.sqrt(jnp.sum(jnp.where(is_s, xsq, 0.0), axis=1, keepdims=True))
    n2 = jnp.sqrt(jnp.sum(jnp.where(is_s, 0.0, xsq), axis=1, keepdims=True))
    scale = jnp.where(is_s,
                      1.0 / jnp.maximum(n1, 1e-12),
                      XY_WEIGHT / jnp.maximum(n2, 1e-12))
    a = sxy * scale
    a2 = a * a

    cr = cr_ref[...].reshape(NB, 2 * M)
    cs = cr[:, :M]
    rk = cr[:, M:]
    tot = jnp.max(cs, axis=1, keepdims=True)
    thr = tp_ref[0, 0] * (tot + 1e-8)
    kcnt = jnp.sum(jnp.where(cs <= thr, 1.0, 0.0), axis=1, keepdims=True)
    kcnt = jnp.maximum(kcnt, 1.0)
    wm = jnp.where(rk <= kcnt, 1.0 / kcnt, 0.0)

    ones_row = jnp.ones((1, D), jnp.float32)
    s2_rows = [jax.lax.dot_general(ones_row, a2[i * M:(i + 1) * M, :],
                                   (((1,), (1,)), ((), ())),
                                   preferred_element_type=jnp.float32)
               for i in range(NB)]
    s2_row = jnp.concatenate(s2_rows, axis=0)
    wm = wm * jnp.exp2((-BETA * LOG2E) * s2_row)

    g = jnp.dot(a * (2.0 * BETA * LOG2E), cft_ref[...],
                preferred_element_type=jnp.float32)
    sim = jnp.exp2(g + cb_ref[...]).astype(jnp.bfloat16)
    wmb = wm.astype(jnp.bfloat16)
    ts = [jnp.dot(wmb[i:i + 1, :], sim[i * M:(i + 1) * M, :],
                  preferred_element_type=jnp.float32)
          for i in range(NB)]
    t = jnp.concatenate(ts, axis=0)
    scores = jnp.dot(t, sel_ref[...],
                     preferred_element_type=jnp.float32)
    alpha = jax.nn.sigmoid(ap_ref[0, 0])
    out_ref[...] = (alpha * base_ref[...]
                    + (1.0 - alpha) * scores[:, 0:C].reshape(NB, 1, C))


@jax.jit
def kernel(base_b, S_slots_b, XY_b, P_b, mask_b, centers, psi, alpha_param,
           top_p):
    f32 = jnp.float32
    pg = P_b.reshape(NG, 16, M).transpose(0, 2, 1)
    mg = mask_b.reshape(NG, 16, M).transpose(0, 2, 1)
    buf = _topp_sc(pg, mg)
    cr_rows = buf.transpose(0, 2, 1).reshape(B, 1, 2 * M)

    cf = centers.reshape(CK, D)
    psif = psi.reshape(1, CK)
    base3 = base_b.reshape(B, 1, C)
    tp = jnp.reshape(top_p.astype(f32), (1, 1))
    ap = jnp.reshape(alpha_param.astype(f32), (1, 1))

    grid = (B // NB,)
    fixed = lambda i: (0, 0)
    out = pl.pallas_call(
        _mixer_kernel,
        grid=grid,
        in_specs=[
            pl.BlockSpec((1, 1), fixed),
            pl.BlockSpec((1, 1), fixed),
            pl.BlockSpec((NB, M, DSLOT), lambda i: (i, 0, 0)),
            pl.BlockSpec((NB, M, 2), lambda i: (i, 0, 0)),
            pl.BlockSpec((NB, 1, 2 * M), lambda i: (i, 0, 0)),
            pl.BlockSpec((NB, 1, C), lambda i: (i, 0, 0)),
            pl.BlockSpec((CK, D), fixed),
            pl.BlockSpec((1, CK), fixed),
        ],
        out_specs=pl.BlockSpec((NB, 1, C), lambda i: (i, 0, 0)),
        out_shape=jax.ShapeDtypeStruct((B, 1, C), f32),
        scratch_shapes=[
            pltpu.VMEM((D, CK), f32),
            pltpu.VMEM((1, CK), f32),
            pltpu.VMEM((CK, D), f32),
        ],
    )(tp, ap, S_slots_b, XY_b, cr_rows, base3, cf, psif)
    return out.reshape(B, C)


# speedup vs baseline: 1.0642x; 1.0642x over previous
import functools

import jax
import jax.numpy as jnp
from jax import lax
from jax.experimental import pallas as pl
from jax.experimental.pallas import tpu as pltpu
from jax.experimental.pallas import tpu_sc as plsc

BETA = 5.0
XY_WEIGHT = 0.5
B, M, DSLOT, C, K, D = 64, 256, 126, 100, 32, 128
CK = C * K
LOG2E = 1.4426950408889634
NB = 4
NG = 4


def _make_sc_body(G):
    TPG = 32 // G
    MJt = M // TPG

    def _topp_sc_body(p_hbm, m_hbm, out_hbm, pv, mv, csv, cnv):
        tile = lax.axis_index("c") * 16 + lax.axis_index("s")
        grp = tile // TPG
        j0 = (tile % TPG) * MJt
        pltpu.sync_copy(p_hbm.at[grp], pv)
        pltpu.sync_copy(m_hbm.at[grp], mv)
        zeros = jnp.zeros((16,), jnp.float32)

        def _smul(j, carry):
            pv[j] = pv[j] * mv[j]
            return carry
        lax.fori_loop(0, M, _smul, zeros)

        def _jbody(jj, carry):
            j = j0 + jj
            sj = pv[j]

            def _lbody(l, acc):
                cs, cn = acc
                sl = pv[l]
                lef = jnp.where(l <= j, 1.0, 0.0)
                m = jnp.where(sl > sj, 1.0, jnp.where(sl == sj, lef, 0.0))
                return (cs + m * sl, cn + m)

            cs, cn = lax.fori_loop(0, M, _lbody, (zeros, zeros), unroll=16)
            csv[jj] = cs
            cnv[jj] = cn
            return carry

        lax.fori_loop(0, MJt, _jbody, zeros)
        pltpu.sync_copy(csv, out_hbm.at[grp, pl.ds(j0, MJt)])
        pltpu.sync_copy(cnv, out_hbm.at[grp, pl.ds(M + j0, MJt)])

    return _topp_sc_body


def _topp_sc(pg, mg):
    f32 = jnp.float32
    G = pg.shape[0]
    MJt = M // (32 // G)
    fn = functools.partial(
        pl.kernel,
        mesh=plsc.VectorSubcoreMesh(core_axis_name="c", subcore_axis_name="s"),
        out_type=jax.ShapeDtypeStruct((G, 2 * M, 16), f32),
        scratch_types=[
            pltpu.VMEM((M, 16), f32),
            pltpu.VMEM((M, 16), f32),
            pltpu.VMEM((MJt, 16), f32),
            pltpu.VMEM((MJt, 16), f32),
        ],
    )(_make_sc_body(G))
    return fn(pg, mg)


def _prep_body(cf_ref, psif_ref, cft_out, cb_out, sel_out):
    cf = cf_ref[...]
    cft_out[...] = cf.T
    cft = cft_out[...]
    c2 = jnp.dot(jnp.ones((1, D), jnp.float32), cft * cft,
                 preferred_element_type=jnp.float32)
    seg = jax.lax.broadcasted_iota(jnp.int32, (CK, D), 0) // K
    cidx = jax.lax.broadcasted_iota(jnp.int32, (CK, D), 1)
    sel = (seg == cidx).astype(jnp.float32)
    sel_out[...] = sel
    psif = psif_ref[...]
    mg = jnp.max(psif)
    e = jnp.exp(psif - mg)
    seg_sum = jnp.dot(e, sel, preferred_element_type=jnp.float32)
    sums = jax.lax.dot_general(seg_sum, sel, (((1,), (1,)), ((), ())),
                               preferred_element_type=jnp.float32)
    lnw = psif - mg - jnp.log(sums)
    cb_out[...] = LOG2E * (lnw - BETA * c2)


def _mixer_kernel(tp_ref, ap_ref, s_ref, xy_ref, cr_ref, base_ref, cf_ref,
                  psif_ref, out_ref, cft_ref, cb_ref, sel_ref):
    @pl.when(pl.program_id(0) == 0)
    def _prep():
        _prep_body(cf_ref, psif_ref, cft_ref, cb_ref, sel_ref)

    MM = NB * M
    s_in = s_ref[...].reshape(MM, DSLOT)
    xy_in = xy_ref[...].reshape(MM, 2)
    sxy = jnp.concatenate([s_in, xy_in], axis=-1)
    xsq = sxy * sxy
    lane = jax.lax.broadcasted_iota(jnp.int32, (MM, D), 1)
    is_s = lane < DSLOT
    n1 = jnp.sqrt(jnp.sum(jnp.where(is_s, xsq, 0.0), axis=1, keepdims=True))
    n2 = jnp.sqrt(jnp.sum(jnp.where(is_s, 0.0, xsq), axis=1, keepdims=True))
    scale = jnp.where(is_s,
                      1.0 / jnp.maximum(n1, 1e-12),
                      XY_WEIGHT / jnp.maximum(n2, 1e-12))
    a = sxy * scale
    a2 = a * a

    cr = cr_ref[...].reshape(NB, 2 * M)
    cs = cr[:, :M]
    rk = cr[:, M:]
    tot = jnp.max(cs, axis=1, keepdims=True)
    thr = tp_ref[0, 0] * (tot + 1e-8)
    kcnt = jnp.sum(jnp.where(cs <= thr, 1.0, 0.0), axis=1, keepdims=True)
    kcnt = jnp.maximum(kcnt, 1.0)
    wm = jnp.where(rk <= kcnt, 1.0 / kcnt, 0.0)

    ones_row = jnp.ones((1, D), jnp.float32)
    s2_rows = [jax.lax.dot_general(ones_row, a2[i * M:(i + 1) * M, :],
                                   (((1,), (1,)), ((), ())),
                                   preferred_element_type=jnp.float32)
               for i in range(NB)]
    s2_row = jnp.concatenate(s2_rows, axis=0)
    wm = wm * jnp.exp2((-BETA * LOG2E) * s2_row)

    g = jnp.dot(a * (2.0 * BETA * LOG2E), cft_ref[...],
                preferred_element_type=jnp.float32)
    sim = jnp.exp2(g + cb_ref[...]).astype(jnp.bfloat16)
    wmb = wm.astype(jnp.bfloat16)
    ts = [jnp.dot(wmb[i:i + 1, :], sim[i * M:(i + 1) * M, :],
                  preferred_element_type=jnp.float32)
          for i in range(NB)]
    t = jnp.concatenate(ts, axis=0)
    scores = jnp.dot(t, sel_ref[...],
                     preferred_element_type=jnp.float32)
    alpha = jax.nn.sigmoid(ap_ref[0, 0])
    out_ref[...] = (alpha * base_ref[...]
                    + (1.0 - alpha) * scores[:, 0:C].reshape(NB, 1, C))


@jax.jit
def kernel(base_b, S_slots_b, XY_b, P_b, mask_b, centers, psi, alpha_param,
           top_p):
    f32 = jnp.float32
    pg = P_b.reshape(NG, 16, M).transpose(0, 2, 1)
    mg = mask_b.reshape(NG, 16, M).transpose(0, 2, 1)
    buf = _topp_sc(pg, mg)
    cr_rows = buf.transpose(0, 2, 1).reshape(B, 1, 2 * M)

    cf = centers.reshape(CK, D)
    psif = psi.reshape(1, CK)
    base3 = base_b.reshape(B, 1, C)
    tp = jnp.reshape(top_p.astype(f32), (1, 1))
    ap = jnp.reshape(alpha_param.astype(f32), (1, 1))

    grid = (B // NB,)
    fixed = lambda i: (0, 0)
    out = pl.pallas_call(
        _mixer_kernel,
        grid=grid,
        in_specs=[
            pl.BlockSpec((1, 1), fixed),
            pl.BlockSpec((1, 1), fixed),
            pl.BlockSpec((NB, M, DSLOT), lambda i: (i, 0, 0)),
            pl.BlockSpec((NB, M, 2), lambda i: (i, 0, 0)),
            pl.BlockSpec((NB, 1, 2 * M), lambda i: (i, 0, 0)),
            pl.BlockSpec((NB, 1, C), lambda i: (i, 0, 0)),
            pl.BlockSpec((CK, D), fixed),
            pl.BlockSpec((1, CK), fixed),
        ],
        out_specs=pl.BlockSpec((NB, 1, C), lambda i: (i, 0, 0)),
        out_shape=jax.ShapeDtypeStruct((B, 1, C), f32),
        scratch_shapes=[
            pltpu.VMEM((D, CK), f32),
            pltpu.VMEM((1, CK), f32),
            pltpu.VMEM((CK, D), f32),
        ],
    )(tp, ap, S_slots_b, XY_b, cr_rows, base3, cf, psif)
    return out.reshape(B, C)
